# baseline (speedup 1.0000x reference)
import functools

import jax
import jax.numpy as jnp
from jax.experimental import pallas as pl
from jax.experimental.pallas import tpu as pltpu
from jax.experimental.pallas import tpu_sc as plsc

_NUM_SAMPLES = 100000
_NUM_FEATURES = 128
_BATCH = 1024
_TEMP = 0.05
_LOG2E = 1.4426950408889634
_CHUNK = 4000
_STEPS = _NUM_SAMPLES // _CHUNK

_NC = 2
_NS = 16
_NW = _NC * _NS
_BPW = _BATCH // _NW


def _gather_body(table_hbm, idx_hbm, out_hbm, idx_v, rows_v, sem):
    wid = jax.lax.axis_index("s") * _NC + jax.lax.axis_index("c")
    base = wid * _BPW
    pltpu.sync_copy(idx_hbm.at[pl.ds(base, _BPW)], idx_v)
    pltpu.async_copy(table_hbm.at[idx_v], rows_v, sem).wait()
    pltpu.sync_copy(rows_v, out_hbm.at[pl.ds(base, _BPW)])


def _sc_gather(features, targets):
    k = functools.partial(
        pl.kernel,
        mesh=plsc.VectorSubcoreMesh(core_axis_name="c", subcore_axis_name="s"),
        out_type=jax.ShapeDtypeStruct((_BATCH, _NUM_FEATURES), jnp.float32),
        scratch_types=[
            pltpu.VMEM((_BPW,), jnp.int32),
            pltpu.VMEM((_BPW, _NUM_FEATURES), jnp.float32),
            pltpu.SemaphoreType.DMA,
        ],
    )(_gather_body)
    return k(features, targets)


def _loss_body(x_ref, tf_ref, f_ref, out_ref, xs_ref, s_ref, tl_ref):
    step = pl.program_id(0)

    @pl.when(step == 0)
    def _init():
        x = x_ref[...]
        xn = x * jax.lax.rsqrt(jnp.sum(x * x, axis=1, keepdims=True))
        xs = xn * (1.0 / _TEMP)
        tl_ref[...] = jnp.sum(xs * tf_ref[...], axis=1, keepdims=True)
        xs_ref[...] = (xn * (_LOG2E / _TEMP)).astype(jnp.bfloat16)
        s_ref[...] = jnp.zeros_like(s_ref)

    logits2 = jax.lax.dot_general(
        xs_ref[...], f_ref[...].astype(jnp.bfloat16), (((1,), (1,)), ((), ())),
        preferred_element_type=jnp.float32,
    )
    s_ref[...] += jnp.sum(jnp.exp2(logits2), axis=1, keepdims=True)

    @pl.when(step == _STEPS - 1)
    def _fin():
        nll = jnp.log(s_ref[...]) - tl_ref[...]
        out_ref[...] = jnp.sum(nll, axis=0, keepdims=True) * (1.0 / _BATCH)


def kernel(inputs, targets, features):
    tgt_feats = _sc_gather(features, targets.astype(jnp.int32))
    out = pl.pallas_call(
        _loss_body,
        grid=(_STEPS,),
        in_specs=[
            pl.BlockSpec((_BATCH, _NUM_FEATURES), lambda i: (0, 0)),
            pl.BlockSpec((_BATCH, _NUM_FEATURES), lambda i: (0, 0)),
            pl.BlockSpec((_CHUNK, _NUM_FEATURES), lambda i: (i, 0)),
        ],
        out_specs=pl.BlockSpec((1, 1), lambda i: (0, 0)),
        out_shape=jax.ShapeDtypeStruct((1, 1), jnp.float32),
        scratch_shapes=[
            pltpu.VMEM((_BATCH, _NUM_FEATURES), jnp.bfloat16),
            pltpu.VMEM((_BATCH, 1), jnp.float32),
            pltpu.VMEM((_BATCH, 1), jnp.float32),
        ],
    )(inputs, tgt_feats, features)
    return out[0, 0]


# speedup vs baseline: 1.0825x; 1.0825x over previous
import functools

import jax
import jax.numpy as jnp
from jax.experimental import pallas as pl
from jax.experimental.pallas import tpu as pltpu
from jax.experimental.pallas import tpu_sc as plsc

_NUM_SAMPLES = 100000
_NUM_FEATURES = 128
_BATCH = 1024
_TEMP = 0.05
_LOG2E = 1.4426950408889634
_CHUNK = 20000
_STEPS = _NUM_SAMPLES // _CHUNK

_NC = 2
_NS = 16
_NW = _NC * _NS
_BPW = _BATCH // _NW


def _gather_body(table_hbm, idx_hbm, out_hbm, idx_v, rows_v, sem):
    wid = jax.lax.axis_index("s") * _NC + jax.lax.axis_index("c")
    base = wid * _BPW
    pltpu.sync_copy(idx_hbm.at[pl.ds(base, _BPW)], idx_v)
    pltpu.async_copy(table_hbm.at[idx_v], rows_v, sem).wait()
    pltpu.sync_copy(rows_v, out_hbm.at[pl.ds(base, _BPW)])


def _sc_gather(features, targets):
    k = functools.partial(
        pl.kernel,
        mesh=plsc.VectorSubcoreMesh(core_axis_name="c", subcore_axis_name="s"),
        out_type=jax.ShapeDtypeStruct((_BATCH, _NUM_FEATURES), jnp.float32),
        scratch_types=[
            pltpu.VMEM((_BPW,), jnp.int32),
            pltpu.VMEM((_BPW, _NUM_FEATURES), jnp.float32),
            pltpu.SemaphoreType.DMA,
        ],
    )(_gather_body)
    return k(features, targets)


def _loss_body(x_ref, tf_ref, f_ref, out_ref, xs_ref, s_ref, tl_ref):
    step = pl.program_id(0)

    @pl.when(step == 0)
    def _init():
        x = x_ref[...]
        xn = x * jax.lax.rsqrt(jnp.sum(x * x, axis=1, keepdims=True))
        xs = xn * (1.0 / _TEMP)
        tl_ref[...] = jnp.sum(xs * tf_ref[...], axis=1, keepdims=True)
        xs_ref[...] = (xn * (_LOG2E / _TEMP)).astype(jnp.bfloat16)
        s_ref[...] = jnp.zeros_like(s_ref)

    logits2 = jax.lax.dot_general(
        xs_ref[...], f_ref[...].astype(jnp.bfloat16), (((1,), (1,)), ((), ())),
        preferred_element_type=jnp.float32,
    )
    s_ref[...] += jnp.sum(jnp.exp2(logits2), axis=1, keepdims=True)

    @pl.when(step == _STEPS - 1)
    def _fin():
        nll = jnp.log(s_ref[...]) - tl_ref[...]
        out_ref[...] = jnp.sum(nll, axis=0, keepdims=True) * (1.0 / _BATCH)


def kernel(inputs, targets, features):
    tgt_feats = _sc_gather(features, targets.astype(jnp.int32))
    out = pl.pallas_call(
        _loss_body,
        grid=(_STEPS,),
        in_specs=[
            pl.BlockSpec((_BATCH, _NUM_FEATURES), lambda i: (0, 0)),
            pl.BlockSpec((_BATCH, _NUM_FEATURES), lambda i: (0, 0)),
            pl.BlockSpec((_CHUNK, _NUM_FEATURES), lambda i: (i, 0)),
        ],
        out_specs=pl.BlockSpec((1, 1), lambda i: (0, 0)),
        out_shape=jax.ShapeDtypeStruct((1, 1), jnp.float32),
        scratch_shapes=[
            pltpu.VMEM((_BATCH, _NUM_FEATURES), jnp.bfloat16),
            pltpu.VMEM((_BATCH, 1), jnp.float32),
            pltpu.VMEM((_BATCH, 1), jnp.float32),
        ],
    )(inputs, tgt_feats, features)
    return out[0, 0]


# speedup vs baseline: 1.1245x; 1.0389x over previous
import functools

import jax
import jax.numpy as jnp
from jax.experimental import pallas as pl
from jax.experimental.pallas import tpu as pltpu
from jax.experimental.pallas import tpu_sc as plsc

_NUM_SAMPLES = 100000
_NUM_FEATURES = 128
_BATCH = 1024
_TEMP = 0.05
_LOG2E = 1.4426950408889634
_CHUNK = 20000
_STEPS = _NUM_SAMPLES // _CHUNK

_NC = 2
_NS = 16
_NW = _NC * _NS
_BPW = _BATCH // _NW


def _gather_body(table_hbm, idx_hbm, out_hbm, idx_v, rows_v, sem):
    wid = jax.lax.axis_index("s") * _NC + jax.lax.axis_index("c")
    base = wid * _BPW
    pltpu.sync_copy(idx_hbm.at[pl.ds(base, _BPW)], idx_v)
    pltpu.async_copy(table_hbm.at[idx_v], rows_v, sem).wait()
    pltpu.sync_copy(rows_v, out_hbm.at[pl.ds(base, _BPW)])


def _sc_gather(features, targets):
    k = functools.partial(
        pl.kernel,
        mesh=plsc.VectorSubcoreMesh(core_axis_name="c", subcore_axis_name="s"),
        out_type=jax.ShapeDtypeStruct((_BATCH, _NUM_FEATURES), jnp.float32),
        scratch_types=[
            pltpu.VMEM((_BPW,), jnp.int32),
            pltpu.VMEM((_BPW, _NUM_FEATURES), jnp.float32),
            pltpu.SemaphoreType.DMA,
        ],
    )(_gather_body)
    return k(features, targets)


def _sumexp_body(x_ref, f_ref, out_ref, xs_ref, s_ref):
    step = pl.program_id(0)

    @pl.when(step == 0)
    def _init():
        x = x_ref[...]
        xn = x * jax.lax.rsqrt(jnp.sum(x * x, axis=1, keepdims=True))
        xs_ref[...] = (xn * (_LOG2E / _TEMP)).astype(jnp.bfloat16)
        s_ref[...] = jnp.zeros_like(s_ref)

    logits2 = jax.lax.dot_general(
        xs_ref[...], f_ref[...].astype(jnp.bfloat16), (((1,), (1,)), ((), ())),
        preferred_element_type=jnp.float32,
    )
    s_ref[...] += jnp.sum(jnp.exp2(logits2), axis=1, keepdims=True)

    @pl.when(step == _STEPS - 1)
    def _fin():
        out_ref[...] = s_ref[...]


def _finish_body(x_ref, tf_ref, s_ref, out_ref):
    x = x_ref[...]
    xs = x * (jax.lax.rsqrt(jnp.sum(x * x, axis=1, keepdims=True))
              * (1.0 / _TEMP))
    tl = jnp.sum(xs * tf_ref[...], axis=1, keepdims=True)
    nll = jnp.log(s_ref[...]) - tl
    out_ref[...] = jnp.sum(nll, axis=0, keepdims=True) * (1.0 / _BATCH)


def kernel(inputs, targets, features):
    tgt_feats = _sc_gather(features, targets.astype(jnp.int32))
    s = pl.pallas_call(
        _sumexp_body,
        grid=(_STEPS,),
        in_specs=[
            pl.BlockSpec((_BATCH, _NUM_FEATURES), lambda i: (0, 0)),
            pl.BlockSpec((_CHUNK, _NUM_FEATURES), lambda i: (i, 0)),
        ],
        out_specs=pl.BlockSpec((_BATCH, 1), lambda i: (0, 0)),
        out_shape=jax.ShapeDtypeStruct((_BATCH, 1), jnp.float32),
        scratch_shapes=[
            pltpu.VMEM((_BATCH, _NUM_FEATURES), jnp.bfloat16),
            pltpu.VMEM((_BATCH, 1), jnp.float32),
        ],
    )(inputs, features)
    out = pl.pallas_call(
        _finish_body,
        out_shape=jax.ShapeDtypeStruct((1, 1), jnp.float32),
    )(inputs, tgt_feats, s)
    return out[0, 0]
